# baseline (speedup 1.0000x reference)
import functools

import jax
import jax.numpy as jnp
from jax import lax
from jax.experimental import pallas as pl
from jax.experimental.pallas import tpu as pltpu
from jax.experimental.pallas import tpu_sc as plsc

_LANES = 16
_CHUNK = 4096
_UNROLL = 16
_RW = 128


@functools.lru_cache(maxsize=None)
def _build(num_fields: int, vocab: int, embed_dim: int, batch: int):
    info = plsc.get_sparse_core_info()
    nc, ns = info.num_cores, info.num_subcores
    nw = nc * ns
    assert embed_dim == nw
    assert batch % _CHUNK == 0
    nchunks = batch // _CHUNK
    assert nchunks % 2 == 0
    total_words = num_fields * embed_dim * vocab
    assert total_words % _RW == 0
    total_rows = total_words // _RW
    nrows = -(-(vocab + _RW - 1 + _RW - 1) // _RW)
    nrows = -(-nrows // 16) * 16
    nfull, nrem = nrows // _RW, nrows % _RW
    mesh = plsc.VectorSubcoreMesh(core_axis_name="c", subcore_axis_name="s")

    @functools.partial(
        pl.kernel,
        mesh=mesh,
        compiler_params=pltpu.CompilerParams(use_tc_tiling_on_sc=False,
                                             needs_layout_passes=False),
        out_type=jax.ShapeDtypeStruct((num_fields * embed_dim, batch),
                                      jnp.float32),
        scratch_types=[
            pltpu.VMEM((nrows, _RW), jnp.float32),
            pltpu.VMEM((nrows,), jnp.int32),
            pltpu.VMEM((2, _CHUNK), jnp.int32),
            pltpu.VMEM((2, _CHUNK), jnp.float32),
            pltpu.SemaphoreType.DMA,
            pltpu.SemaphoreType.DMA,
            pltpu.SemaphoreType.DMA,
        ],
    )
    def emb_kernel(xt_hbm, tab_hbm, out_hbm, row_v, ridx_v, xbuf, obuf,
                   rsem, xsem, wsem):
        e = lax.axis_index("s") * nc + lax.axis_index("c")
        lane = lax.iota(jnp.int32, 16)

        def row_desc(qo, qn):
            return pltpu.make_async_copy(
                tab_hbm.at[ridx_v.at[pl.ds(qo, qn)]],
                row_v.at[pl.ds(qo, qn)],
                rsem,
            )

        def qspans():
            spans = [(q * _RW, _RW) for q in range(nfull)]
            if nrem:
                spans.append((nfull * _RW, nrem))
            return spans

        def field_body(f, carry):
            orow = f * embed_dim + e
            w0 = orow * vocab
            r0 = w0 // _RW
            off = w0 % _RW

            def rix(i, carry2):
                sl = pl.ds(i * _LANES, _LANES)
                ridx_v[sl] = jnp.minimum(r0 + i * _LANES + lane,
                                         total_rows - 1)
                return carry2

            lax.fori_loop(0, nrows // _LANES, rix, 0)

            for qo, qn in qspans():
                row_desc(qo, qn).start()
            for qo, qn in qspans():
                row_desc(qo, qn).wait()

            pltpu.async_copy(xt_hbm.at[f, pl.ds(0, _CHUNK)], xbuf.at[0],
                             xsem).wait()

            for ch in range(nchunks):
                p = ch % 2
                if ch + 1 < nchunks:
                    nxt = pltpu.async_copy(
                        xt_hbm.at[f, pl.ds((ch + 1) * _CHUNK, _CHUNK)],
                        xbuf.at[1 - p], xsem)
                if ch >= 2:
                    pltpu.make_async_copy(
                        obuf.at[p],
                        out_hbm.at[orow, pl.ds((ch - 2) * _CHUNK, _CHUNK)],
                        wsem,
                    ).wait()

                def gat(i, carry2):
                    for u in range(_UNROLL):
                        sl = pl.ds((i * _UNROLL + u) * _LANES, _LANES)
                        local = xbuf[p, sl] + off
                        obuf[p, sl] = plsc.load_gather(
                            row_v,
                            [lax.shift_right_logical(local, 7),
                             lax.bitwise_and(local, _RW - 1)],
                        )
                    return carry2

                lax.fori_loop(0, _CHUNK // (_LANES * _UNROLL), gat, 0)
                pltpu.make_async_copy(
                    obuf.at[p],
                    out_hbm.at[orow, pl.ds(ch * _CHUNK, _CHUNK)],
                    wsem,
                ).start()
                if ch + 1 < nchunks:
                    nxt.wait()

            for p in (0, 1):
                pltpu.make_async_copy(
                    obuf.at[p],
                    out_hbm.at[orow, pl.ds((nchunks - 2 + p) * _CHUNK,
                                           _CHUNK)],
                    wsem,
                ).wait()
            return carry

        lax.fori_loop(0, num_fields, field_body, 0)

    return emb_kernel


def kernel(x, tables):
    batch, num_fields = x.shape
    nf, vocab, embed_dim = tables.shape
    assert nf == num_fields
    emb = _build(num_fields, vocab, embed_dim, batch)
    flat_rows = tables.transpose(0, 2, 1).reshape(-1, _RW)
    out_t = emb(x.T, flat_rows)
    return out_t.T.reshape(batch, num_fields * embed_dim)


# speedup vs baseline: 1.0039x; 1.0039x over previous
import functools

import jax
import jax.numpy as jnp
from jax import lax
from jax.experimental import pallas as pl
from jax.experimental.pallas import tpu as pltpu
from jax.experimental.pallas import tpu_sc as plsc

_LANES = 16
_CHUNK = 4096
_UNROLL = 16
_RW = 1024
_SHIFT = 10


@functools.lru_cache(maxsize=None)
def _build(num_fields: int, vocab: int, embed_dim: int, batch: int):
    info = plsc.get_sparse_core_info()
    nc, ns = info.num_cores, info.num_subcores
    nw = nc * ns
    assert embed_dim == nw
    assert batch % _CHUNK == 0
    nchunks = batch // _CHUNK
    assert nchunks % 2 == 0
    total_words = num_fields * embed_dim * vocab
    assert total_words % _RW == 0
    total_rows = total_words // _RW
    nrows = -(-(vocab + _RW - 1 + _RW - 1) // _RW)
    nidx = -(-nrows // 16) * 16
    mesh = plsc.VectorSubcoreMesh(core_axis_name="c", subcore_axis_name="s")

    @functools.partial(
        pl.kernel,
        mesh=mesh,
        compiler_params=pltpu.CompilerParams(use_tc_tiling_on_sc=False,
                                             needs_layout_passes=False),
        out_type=jax.ShapeDtypeStruct((num_fields * embed_dim, batch),
                                      jnp.float32),
        scratch_types=[
            pltpu.VMEM((nrows, _RW), jnp.float32),
            pltpu.VMEM((nidx,), jnp.int32),
            pltpu.VMEM((2, _CHUNK), jnp.int32),
            pltpu.VMEM((2, _CHUNK), jnp.float32),
            pltpu.SemaphoreType.DMA,
            pltpu.SemaphoreType.DMA,
            pltpu.SemaphoreType.DMA,
        ],
    )
    def emb_kernel(xt_hbm, tab_hbm, out_hbm, row_v, ridx_v, xbuf, obuf,
                   rsem, xsem, wsem):
        e = lax.axis_index("s") * nc + lax.axis_index("c")
        lane = lax.iota(jnp.int32, 16)

        def row_desc(qo, qn):
            return pltpu.make_async_copy(
                tab_hbm.at[ridx_v.at[pl.ds(qo, qn)]],
                row_v.at[pl.ds(qo, qn)],
                rsem,
            )

        def qspans():
            return [(0, nrows)]

        def field_body(f, carry):
            orow = f * embed_dim + e
            w0 = orow * vocab
            r0 = w0 // _RW
            off = w0 % _RW

            def rix(i, carry2):
                sl = pl.ds(i * _LANES, _LANES)
                ridx_v[sl] = jnp.minimum(r0 + i * _LANES + lane,
                                         total_rows - 1)
                return carry2

            lax.fori_loop(0, nidx // _LANES, rix, 0)

            for qo, qn in qspans():
                row_desc(qo, qn).start()
            for qo, qn in qspans():
                row_desc(qo, qn).wait()

            pltpu.async_copy(xt_hbm.at[f, pl.ds(0, _CHUNK)], xbuf.at[0],
                             xsem).wait()

            for ch in range(nchunks):
                p = ch % 2
                if ch + 1 < nchunks:
                    nxt = pltpu.async_copy(
                        xt_hbm.at[f, pl.ds((ch + 1) * _CHUNK, _CHUNK)],
                        xbuf.at[1 - p], xsem)
                if ch >= 2:
                    pltpu.make_async_copy(
                        obuf.at[p],
                        out_hbm.at[orow, pl.ds((ch - 2) * _CHUNK, _CHUNK)],
                        wsem,
                    ).wait()

                def gat(i, carry2):
                    for u in range(_UNROLL):
                        sl = pl.ds((i * _UNROLL + u) * _LANES, _LANES)
                        local = xbuf[p, sl] + off
                        obuf[p, sl] = plsc.load_gather(
                            row_v,
                            [lax.shift_right_logical(local, _SHIFT),
                             lax.bitwise_and(local, _RW - 1)],
                        )
                    return carry2

                lax.fori_loop(0, _CHUNK // (_LANES * _UNROLL), gat, 0)
                pltpu.make_async_copy(
                    obuf.at[p],
                    out_hbm.at[orow, pl.ds(ch * _CHUNK, _CHUNK)],
                    wsem,
                ).start()
                if ch + 1 < nchunks:
                    nxt.wait()

            for p in (0, 1):
                pltpu.make_async_copy(
                    obuf.at[p],
                    out_hbm.at[orow, pl.ds((nchunks - 2 + p) * _CHUNK,
                                           _CHUNK)],
                    wsem,
                ).wait()
            return carry

        lax.fori_loop(0, num_fields, field_body, 0)

    return emb_kernel


def kernel(x, tables):
    batch, num_fields = x.shape
    nf, vocab, embed_dim = tables.shape
    assert nf == num_fields
    emb = _build(num_fields, vocab, embed_dim, batch)
    flat_rows = tables.transpose(0, 2, 1).reshape(-1, _RW)
    out_t = emb(x.T, flat_rows)
    return out_t.T.reshape(batch, num_fields * embed_dim)


# speedup vs baseline: 1.0067x; 1.0028x over previous
import functools

import jax
import jax.numpy as jnp
from jax import lax
from jax.experimental import pallas as pl
from jax.experimental.pallas import tpu as pltpu
from jax.experimental.pallas import tpu_sc as plsc

_LANES = 16
_CHUNK = 4096
_UNROLL = 16


@functools.lru_cache(maxsize=None)
def _build(num_fields: int, vocab: int, embed_dim: int, batch: int):
    info = plsc.get_sparse_core_info()
    nc, ns = info.num_cores, info.num_subcores
    nw = nc * ns
    assert embed_dim == nw
    assert batch % _CHUNK == 0
    nchunks = batch // _CHUNK
    assert nchunks % 2 == 0
    mesh = plsc.VectorSubcoreMesh(core_axis_name="c", subcore_axis_name="s")

    @functools.partial(
        pl.kernel,
        mesh=mesh,
        compiler_params=pltpu.CompilerParams(use_tc_tiling_on_sc=False,
                                             needs_layout_passes=False),
        out_type=jax.ShapeDtypeStruct((num_fields * embed_dim, batch),
                                      jnp.float32),
        scratch_types=[
            pltpu.VMEM((vocab,), jnp.float32),
            pltpu.VMEM((2, _CHUNK), jnp.int32),
            pltpu.VMEM((2, _CHUNK), jnp.float32),
            pltpu.SemaphoreType.DMA,
            pltpu.SemaphoreType.DMA,
            pltpu.SemaphoreType.DMA,
        ],
    )
    def emb_kernel(xt_hbm, tabt_hbm, out_hbm, row_v, xbuf, obuf, xsem, wsem,
                   rsem):
        e = lax.axis_index("s") * nc + lax.axis_index("c")
        nq = 4
        qlen = vocab // nq

        def field_body(f, carry):
            qcopies = [
                pltpu.async_copy(
                    tabt_hbm.at[f, e, pl.ds(q * qlen, qlen)],
                    row_v.at[pl.ds(q * qlen, qlen)],
                    rsem,
                )
                for q in range(nq)
            ]
            for qc in qcopies:
                qc.wait()
            orow = f * embed_dim + e
            pltpu.async_copy(xt_hbm.at[f, pl.ds(0, _CHUNK)], xbuf.at[0],
                             xsem).wait()

            for c in range(nchunks):
                p = c % 2
                if c + 1 < nchunks:
                    nxt = pltpu.async_copy(
                        xt_hbm.at[f, pl.ds((c + 1) * _CHUNK, _CHUNK)],
                        xbuf.at[1 - p], xsem)
                if c >= 2:
                    pltpu.make_async_copy(
                        obuf.at[p],
                        out_hbm.at[orow, pl.ds((c - 2) * _CHUNK, _CHUNK)],
                        wsem,
                    ).wait()

                def gat(i, carry2):
                    for u in range(_UNROLL):
                        sl = pl.ds((i * _UNROLL + u) * _LANES, _LANES)
                        obuf[p, sl] = plsc.load_gather(row_v, [xbuf[p, sl]])
                    return carry2

                lax.fori_loop(0, _CHUNK // (_LANES * _UNROLL), gat, 0)
                pltpu.make_async_copy(
                    obuf.at[p],
                    out_hbm.at[orow, pl.ds(c * _CHUNK, _CHUNK)],
                    wsem,
                ).start()
                if c + 1 < nchunks:
                    nxt.wait()

            for p in (0, 1):
                pltpu.make_async_copy(
                    obuf.at[p],
                    out_hbm.at[orow, pl.ds((nchunks - 2 + p) * _CHUNK,
                                           _CHUNK)],
                    wsem,
                ).wait()
            return carry

        lax.fori_loop(0, num_fields, field_body, 0)

    return emb_kernel


def kernel(x, tables):
    batch, num_fields = x.shape
    nf, vocab, embed_dim = tables.shape
    assert nf == num_fields
    emb = _build(num_fields, vocab, embed_dim, batch)
    out_t = emb(x.T, tables.transpose(0, 2, 1))
    return out_t.T.reshape(batch, num_fields * embed_dim)


# speedup vs baseline: 1.0319x; 1.0251x over previous
import functools

import jax
import jax.numpy as jnp
from jax import lax
from jax.experimental import pallas as pl
from jax.experimental.pallas import tpu as pltpu
from jax.experimental.pallas import tpu_sc as plsc

_LANES = 16
_CHUNK = 4096
_UNROLL = 16


@functools.lru_cache(maxsize=None)
def _build(num_fields: int, vocab: int, embed_dim: int, batch: int):
    info = plsc.get_sparse_core_info()
    nc, ns = info.num_cores, info.num_subcores
    nw = nc * ns
    assert embed_dim == nw
    assert batch % _CHUNK == 0
    nchunks = batch // _CHUNK
    assert nchunks % 2 == 0
    mesh = plsc.VectorSubcoreMesh(core_axis_name="c", subcore_axis_name="s")

    @functools.partial(
        pl.kernel,
        mesh=mesh,
        compiler_params=pltpu.CompilerParams(use_tc_tiling_on_sc=False,
                                             needs_layout_passes=False),
        out_type=jax.ShapeDtypeStruct((num_fields * embed_dim, batch),
                                      jnp.float32),
        scratch_types=[
            pltpu.VMEM((vocab,), jnp.float32),
            pltpu.VMEM((2, _CHUNK), jnp.int32),
            pltpu.VMEM((2, _CHUNK), jnp.float32),
            pltpu.SemaphoreType.DMA,
            pltpu.SemaphoreType.DMA,
            pltpu.SemaphoreType.DMA,
        ],
    )
    def emb_kernel(xt_hbm, tabt_hbm, out_hbm, row_v, xbuf, obuf, xsem, wsem,
                   rsem):
        e = lax.axis_index("s") * nc + lax.axis_index("c")
        nq = 4
        qlen = vocab // nq

        def field_body(f, carry):
            xfirst = pltpu.async_copy(xt_hbm.at[f, pl.ds(0, _CHUNK)],
                                      xbuf.at[0], xsem)
            qcopies = [
                pltpu.async_copy(
                    tabt_hbm.at[f, e, pl.ds(q * qlen, qlen)],
                    row_v.at[pl.ds(q * qlen, qlen)],
                    rsem,
                )
                for q in range(nq)
            ]
            for qc in qcopies:
                qc.wait()
            orow = f * embed_dim + e
            xfirst.wait()

            for c in range(nchunks):
                p = c % 2
                if c + 1 < nchunks:
                    nxt = pltpu.async_copy(
                        xt_hbm.at[f, pl.ds((c + 1) * _CHUNK, _CHUNK)],
                        xbuf.at[1 - p], xsem)
                if c >= 2:
                    pltpu.make_async_copy(
                        obuf.at[p],
                        out_hbm.at[orow, pl.ds((c - 2) * _CHUNK, _CHUNK)],
                        wsem,
                    ).wait()

                def gat(i, carry2):
                    for u in range(_UNROLL):
                        sl = pl.ds((i * _UNROLL + u) * _LANES, _LANES)
                        obuf[p, sl] = plsc.load_gather(row_v, [xbuf[p, sl]])
                    return carry2

                lax.fori_loop(0, _CHUNK // (_LANES * _UNROLL), gat, 0)
                pltpu.make_async_copy(
                    obuf.at[p],
                    out_hbm.at[orow, pl.ds(c * _CHUNK, _CHUNK)],
                    wsem,
                ).start()
                if c + 1 < nchunks:
                    nxt.wait()

            for p in (0, 1):
                pltpu.make_async_copy(
                    obuf.at[p],
                    out_hbm.at[orow, pl.ds((nchunks - 2 + p) * _CHUNK,
                                           _CHUNK)],
                    wsem,
                ).wait()
            return carry

        lax.fori_loop(0, num_fields, field_body, 0)

    return emb_kernel


def kernel(x, tables):
    batch, num_fields = x.shape
    nf, vocab, embed_dim = tables.shape
    assert nf == num_fields
    emb = _build(num_fields, vocab, embed_dim, batch)
    out_t = emb(x.T, tables.transpose(0, 2, 1))
    return out_t.T.reshape(batch, num_fields * embed_dim)
